# baseline (device time: 20137 ns/iter reference)
import jax
import jax.numpy as jnp
from jax import lax
from jax.experimental import pallas as pl
from jax.experimental.pallas import tpu as pltpu

N_DEV = 8
B = 2
SQ = 256
DMODEL = 512
DOUT = 512
HQ = 4
DH = 64
HD = HQ * DH
SKV_SH = 256
WIN = 128
REL1 = SQ + WIN - SKV_SH
SCALE = 0.125
RH = SQ // 2
CHUNKS = [(b, r) for b in range(B) for r in (0, RH)]
NC = len(CHUNKS)


def kernel(x, Wq, K_ext, V_ext, Wo):
    x = x.astype(jnp.bfloat16)
    Wq = Wq.astype(jnp.bfloat16)
    K2 = K_ext.reshape(B, SKV_SH, HD).astype(jnp.bfloat16)
    V2 = V_ext.reshape(B, SKV_SH, HD).astype(jnp.bfloat16)

    def body(x_ref, wq_ref, k_ref, v_ref, wo_ref, out_ref,
             ctx_scr, ctx_rcv, l_snd, l_rcv, p1_send, p1_recv,
             bc_send, bc_recv):
        pos = lax.axis_index("i")
        peer = 1 - pos

        barrier = pltpu.get_barrier_semaphore()
        partner_sets = {0: (1, 3, 4, 6), 1: (0, 2, 5, 7), 2: (1,),
                        3: (0,), 4: (0,), 5: (1,), 6: (0,), 7: (1,)}
        for p, partners in partner_sets.items():
            @pl.when(pos == p)
            def _(partners=partners):
                for t in partners:
                    pl.semaphore_signal(
                        barrier, inc=1, device_id=(t,),
                        device_id_type=pl.DeviceIdType.MESH)
        n_partners = jnp.where(pos <= 1, 4, 1)
        pl.semaphore_wait(barrier, n_partners)

        def exchange_descr(c):
            b, r = CHUNKS[c]
            xl = pltpu.make_async_remote_copy(
                src_ref=l_snd.at[b, pl.ds(r, RH)],
                dst_ref=l_rcv.at[b, pl.ds(r, RH)],
                send_sem=p1_send.at[1, c], recv_sem=p1_recv.at[1, c],
                device_id=(peer,), device_id_type=pl.DeviceIdType.MESH)
            xc = pltpu.make_async_remote_copy(
                src_ref=ctx_scr.at[b, pl.ds(r, RH)],
                dst_ref=ctx_rcv.at[b, pl.ds(r, RH)],
                send_sem=p1_send.at[0, c], recv_sem=p1_recv.at[0, c],
                device_id=(peer,), device_id_type=pl.DeviceIdType.MESH)
            return xl, xc

        def partial_chunk(b, r, kv_len, off):
            qi = lax.broadcasted_iota(jnp.int32, (RH, kv_len), 0) + r
            kj = lax.broadcasted_iota(jnp.int32, (RH, kv_len), 1) + off
            keep = jnp.abs(qi - kj) <= WIN
            q2d = jnp.dot(x_ref[b, r:r + RH], wq_ref[...],
                          preferred_element_type=jnp.float32)
            q_bf = q2d.astype(jnp.bfloat16)
            for h in range(HQ):
                cs = slice(h * DH, (h + 1) * DH)
                s = lax.dot_general(
                    q_bf[:, cs], k_ref[b, 0:kv_len, cs],
                    (((1,), (1,)), ((), ())),
                    preferred_element_type=jnp.float32) * SCALE
                w = jnp.where(keep, jnp.exp(s), 0.0)
                l_snd[b, r:r + RH, h:h + 1] = jnp.sum(w, axis=1,
                                                      keepdims=True)
                ctx_scr[b, r:r + RH, cs] = jnp.dot(
                    w.astype(jnp.bfloat16), v_ref[b, 0:kv_len, cs],
                    preferred_element_type=jnp.float32).astype(jnp.bfloat16)

        @pl.when(pos == 1)
        def _():
            for c, (b, r) in enumerate(CHUNKS):
                if r == RH:
                    partial_chunk(b, r, REL1, SKV_SH)
                    xl, xc = exchange_descr(c)
                    xl.start()
                    xc.start()

        @pl.when(pos == 0)
        def _():
            for c, (b, r) in enumerate(CHUNKS):
                partial_chunk(b, r, SKV_SH, 0)
                xl, xc = exchange_descr(c)
                xl.start()
                xc.start()
                if r == 0:
                    rs = pl.ds(r, RH)
                    for h in range(HQ):
                        cs = slice(h * DH, (h + 1) * DH)
                        ctx_rcv[b, rs, cs] = (
                            ctx_scr[b, rs, cs].astype(jnp.float32)
                            / l_snd[b, rs, h:h + 1]).astype(jnp.bfloat16)

        has_recv = pos >= 2
        recv_from = jnp.where(
            jnp.logical_or(pos == 2,
                           jnp.logical_or(pos == 5, pos == 7)), 1, 0)
        child0 = jnp.where(pos == 0, 3, 2)
        child1 = jnp.where(pos == 0, 4, 5)
        child2 = jnp.where(pos == 0, 6, 7)

        def bc_descr(c, sender_slot, target, src):
            b, r = CHUNKS[c]
            return pltpu.make_async_remote_copy(
                src_ref=src.at[b, pl.ds(r, RH)],
                dst_ref=ctx_scr.at[b, pl.ds(r, RH)],
                send_sem=bc_send.at[sender_slot, c],
                recv_sem=bc_recv.at[c],
                device_id=(target,), device_id_type=pl.DeviceIdType.MESH)

        for c, (b, r) in enumerate(CHUNKS):
            rs = pl.ds(r, RH)

            if r == RH:
                @pl.when(pos == 0)
                def _():
                    xl2, xc2 = exchange_descr(c)
                    xl2.wait_recv()
                    xc2.wait_recv()
                    for h in range(HQ):
                        cs = slice(h * DH, (h + 1) * DH)
                        lt = l_snd[b, rs, h:h + 1] + l_rcv[b, rs, h:h + 1]
                        ctx_rcv[b, rs, cs] = (
                            (ctx_scr[b, rs, cs].astype(jnp.float32)
                             + ctx_rcv[b, rs, cs].astype(jnp.float32))
                            / lt).astype(jnp.bfloat16)

            @pl.when(pos == 1)
            def _():
                xl, xc = exchange_descr(c)
                xl.wait_recv()
                xc.wait_recv()
                if r == 0:
                    for h in range(HQ):
                        cs = slice(h * DH, (h + 1) * DH)
                        ctx_rcv[b, rs, cs] = (
                            ctx_rcv[b, rs, cs].astype(jnp.float32)
                            / l_rcv[b, rs, h:h + 1]).astype(jnp.bfloat16)
                else:
                    for h in range(HQ):
                        cs = slice(h * DH, (h + 1) * DH)
                        lt = l_snd[b, rs, h:h + 1] + l_rcv[b, rs, h:h + 1]
                        ctx_rcv[b, rs, cs] = (
                            (ctx_scr[b, rs, cs].astype(jnp.float32)
                             + ctx_rcv[b, rs, cs].astype(jnp.float32))
                            / lt).astype(jnp.bfloat16)

            @pl.when(has_recv)
            def _():
                bc_descr(c, 0, recv_from, ctx_scr).wait_recv()

            @pl.when(pos <= 1)
            def _():
                bc_descr(c, 0, child0, ctx_rcv).start()
                bc_descr(c, 1, child1, ctx_rcv).start()
                bc_descr(c, 2, child2, ctx_rcv).start()

            @pl.when(has_recv)
            def _():
                out_ref[b, r:r + RH] = jnp.dot(
                    ctx_scr[b, r:r + RH].astype(jnp.float32), wo_ref[...],
                    preferred_element_type=jnp.float32)

        @pl.when(pos <= 1)
        def _():
            for b in range(B):
                out_ref[b] = jnp.dot(ctx_rcv[b].astype(jnp.float32),
                                     wo_ref[...],
                                     preferred_element_type=jnp.float32)

        for c, (b, r) in enumerate(CHUNKS):
            @pl.when(pos == 0)
            def _():
                xl, xc = exchange_descr(c)
                xl.wait_send()
                xc.wait_send()

            @pl.when(pos == 1)
            def _():
                if r == RH:
                    xl, xc = exchange_descr(c)
                    xl.wait_send()
                    xc.wait_send()

            @pl.when(pos <= 1)
            def _():
                bc_descr(c, 0, child0, ctx_rcv).wait_send()
                bc_descr(c, 1, child1, ctx_rcv).wait_send()
                bc_descr(c, 2, child2, ctx_rcv).wait_send()

    return pl.pallas_call(
        body,
        out_shape=jax.ShapeDtypeStruct((B, SQ, DOUT), jnp.float32),
        in_specs=[pl.BlockSpec(memory_space=pltpu.VMEM)] * 5,
        out_specs=pl.BlockSpec(memory_space=pltpu.VMEM),
        scratch_shapes=[
            pltpu.VMEM((B, SQ, HD), jnp.bfloat16),
            pltpu.VMEM((B, SQ, HD), jnp.bfloat16),
            pltpu.VMEM((B, SQ, HQ), jnp.float32),
            pltpu.VMEM((B, SQ, HQ), jnp.float32),
            pltpu.SemaphoreType.DMA((2, NC)),
            pltpu.SemaphoreType.DMA((2, NC)),
            pltpu.SemaphoreType.DMA((3, NC)),
            pltpu.SemaphoreType.DMA((NC,)),
        ],
        compiler_params=pltpu.CompilerParams(collective_id=0),
    )(x, Wq, K2, V2, Wo)


# device time: 17291 ns/iter; 1.1646x vs baseline; 1.1646x over previous
import jax
import jax.numpy as jnp
from jax import lax
from jax.experimental import pallas as pl
from jax.experimental.pallas import tpu as pltpu

N_DEV = 8
B = 2
SQ = 256
DMODEL = 512
DOUT = 512
HQ = 4
DH = 64
HD = HQ * DH
SKV_SH = 256
WIN = 128
REL1 = SQ + WIN - SKV_SH
SCALE = 0.125
RH = SQ // 2
CHUNKS = [(b, r) for b in range(B) for r in (0, RH)]
NC = len(CHUNKS)


def kernel(x, Wq, K_ext, V_ext, Wo):
    x = x.astype(jnp.bfloat16)
    Wq = Wq.astype(jnp.bfloat16)
    K2 = K_ext.reshape(B, SKV_SH, HD).astype(jnp.bfloat16)
    V2 = V_ext.reshape(B, SKV_SH, HD).astype(jnp.bfloat16)

    def body(x_ref, wq_ref, k_ref, v_ref, wo_ref, out_ref,
             ctx_scr, ctx_rcv, l_snd, l_rcv, p1_send, p1_recv,
             bc_send, bc_recv):
        pos = lax.axis_index("i")
        peer = 1 - pos

        barrier = pltpu.get_barrier_semaphore()
        partner_sets = {0: (1, 3, 4), 1: (0, 2, 5), 2: (1, 6),
                        3: (0, 7), 4: (0,), 5: (1,), 6: (2,), 7: (3,)}
        for p, partners in partner_sets.items():
            @pl.when(pos == p)
            def _(partners=partners):
                for t in partners:
                    pl.semaphore_signal(
                        barrier, inc=1, device_id=(t,),
                        device_id_type=pl.DeviceIdType.MESH)
        n_partners = jnp.where(pos <= 1, 3, jnp.where(pos <= 3, 2, 1))
        pl.semaphore_wait(barrier, n_partners)

        def exchange_descr(c):
            b, r = CHUNKS[c]
            xl = pltpu.make_async_remote_copy(
                src_ref=l_snd.at[b, pl.ds(r, RH)],
                dst_ref=l_rcv.at[b, pl.ds(r, RH)],
                send_sem=p1_send.at[1, c], recv_sem=p1_recv.at[1, c],
                device_id=(peer,), device_id_type=pl.DeviceIdType.MESH)
            xc = pltpu.make_async_remote_copy(
                src_ref=ctx_scr.at[b, pl.ds(r, RH)],
                dst_ref=ctx_rcv.at[b, pl.ds(r, RH)],
                send_sem=p1_send.at[0, c], recv_sem=p1_recv.at[0, c],
                device_id=(peer,), device_id_type=pl.DeviceIdType.MESH)
            return xl, xc

        def partial_chunk(b, r, kv_len, off):
            qi = lax.broadcasted_iota(jnp.int32, (RH, kv_len), 0) + r
            kj = lax.broadcasted_iota(jnp.int32, (RH, kv_len), 1) + off
            keep = jnp.abs(qi - kj) <= WIN
            q2d = jnp.dot(x_ref[b, r:r + RH], wq_ref[...],
                          preferred_element_type=jnp.float32)
            q_bf = q2d.astype(jnp.bfloat16)
            for h in range(HQ):
                cs = slice(h * DH, (h + 1) * DH)
                s = lax.dot_general(
                    q_bf[:, cs], k_ref[b, 0:kv_len, cs],
                    (((1,), (1,)), ((), ())),
                    preferred_element_type=jnp.float32) * SCALE
                w = jnp.where(keep, jnp.exp(s), 0.0)
                l_snd[b, r:r + RH, h:h + 1] = jnp.sum(w, axis=1,
                                                      keepdims=True)
                ctx_scr[b, r:r + RH, cs] = jnp.dot(
                    w.astype(jnp.bfloat16), v_ref[b, 0:kv_len, cs],
                    preferred_element_type=jnp.float32).astype(jnp.bfloat16)

        @pl.when(pos == 1)
        def _():
            for c, (b, r) in enumerate(CHUNKS):
                if r == RH:
                    partial_chunk(b, r, REL1, SKV_SH)
                    xl, xc = exchange_descr(c)
                    xl.start()
                    xc.start()

        @pl.when(pos == 0)
        def _():
            for c, (b, r) in enumerate(CHUNKS):
                partial_chunk(b, r, SKV_SH, 0)
                xl, xc = exchange_descr(c)
                xl.start()
                xc.start()
                if r == 0:
                    rs = pl.ds(r, RH)
                    for h in range(HQ):
                        cs = slice(h * DH, (h + 1) * DH)
                        ctx_rcv[b, rs, cs] = (
                            ctx_scr[b, rs, cs].astype(jnp.float32)
                            / l_snd[b, rs, h:h + 1]).astype(jnp.bfloat16)

        has_recv = pos >= 2
        is_fwd = jnp.logical_or(pos == 2, pos == 3)
        recv_from = jnp.where(
            jnp.logical_or(pos == 3, pos == 4), 0,
            jnp.where(jnp.logical_or(pos == 2, pos == 5), 1,
                      jnp.where(pos == 6, 2, 3)))
        child0 = jnp.where(pos == 0, 3,
                           jnp.where(pos == 1, 2,
                                     jnp.where(pos == 2, 6, 7)))
        child1 = jnp.where(pos == 0, 4, 5)

        def bc_descr(c, sender_slot, target, src):
            b, r = CHUNKS[c]
            return pltpu.make_async_remote_copy(
                src_ref=src.at[b, pl.ds(r, RH)],
                dst_ref=ctx_scr.at[b, pl.ds(r, RH)],
                send_sem=bc_send.at[sender_slot, c],
                recv_sem=bc_recv.at[c],
                device_id=(target,), device_id_type=pl.DeviceIdType.MESH)

        for c, (b, r) in enumerate(CHUNKS):
            rs = pl.ds(r, RH)

            if r == RH:
                @pl.when(pos == 0)
                def _():
                    xl2, xc2 = exchange_descr(c)
                    xl2.wait_recv()
                    xc2.wait_recv()
                    for h in range(HQ):
                        cs = slice(h * DH, (h + 1) * DH)
                        lt = l_snd[b, rs, h:h + 1] + l_rcv[b, rs, h:h + 1]
                        ctx_rcv[b, rs, cs] = (
                            (ctx_scr[b, rs, cs].astype(jnp.float32)
                             + ctx_rcv[b, rs, cs].astype(jnp.float32))
                            / lt).astype(jnp.bfloat16)

            @pl.when(pos == 1)
            def _():
                xl, xc = exchange_descr(c)
                xl.wait_recv()
                xc.wait_recv()
                if r == 0:
                    for h in range(HQ):
                        cs = slice(h * DH, (h + 1) * DH)
                        ctx_rcv[b, rs, cs] = (
                            ctx_rcv[b, rs, cs].astype(jnp.float32)
                            / l_rcv[b, rs, h:h + 1]).astype(jnp.bfloat16)
                else:
                    for h in range(HQ):
                        cs = slice(h * DH, (h + 1) * DH)
                        lt = l_snd[b, rs, h:h + 1] + l_rcv[b, rs, h:h + 1]
                        ctx_rcv[b, rs, cs] = (
                            (ctx_scr[b, rs, cs].astype(jnp.float32)
                             + ctx_rcv[b, rs, cs].astype(jnp.float32))
                            / lt).astype(jnp.bfloat16)

            @pl.when(has_recv)
            def _():
                bc_descr(c, 0, recv_from, ctx_scr).wait_recv()

            @pl.when(pos <= 1)
            def _():
                bc_descr(c, 0, child0, ctx_rcv).start()
                bc_descr(c, 1, child1, ctx_rcv).start()

            @pl.when(is_fwd)
            def _():
                bc_descr(c, 0, child0, ctx_scr).start()

            @pl.when(has_recv)
            def _():
                out_ref[b, r:r + RH] = jnp.dot(
                    ctx_scr[b, r:r + RH].astype(jnp.float32), wo_ref[...],
                    preferred_element_type=jnp.float32)

        @pl.when(pos <= 1)
        def _():
            for b in range(B):
                out_ref[b] = jnp.dot(ctx_rcv[b].astype(jnp.float32),
                                     wo_ref[...],
                                     preferred_element_type=jnp.float32)

        for c, (b, r) in enumerate(CHUNKS):
            @pl.when(pos == 0)
            def _():
                xl, xc = exchange_descr(c)
                xl.wait_send()
                xc.wait_send()

            @pl.when(pos == 1)
            def _():
                if r == RH:
                    xl, xc = exchange_descr(c)
                    xl.wait_send()
                    xc.wait_send()

            @pl.when(pos <= 1)
            def _():
                bc_descr(c, 0, child0, ctx_rcv).wait_send()
                bc_descr(c, 1, child1, ctx_rcv).wait_send()

            @pl.when(is_fwd)
            def _():
                bc_descr(c, 0, child0, ctx_scr).wait_send()

    return pl.pallas_call(
        body,
        out_shape=jax.ShapeDtypeStruct((B, SQ, DOUT), jnp.float32),
        in_specs=[pl.BlockSpec(memory_space=pltpu.VMEM)] * 5,
        out_specs=pl.BlockSpec(memory_space=pltpu.VMEM),
        scratch_shapes=[
            pltpu.VMEM((B, SQ, HD), jnp.bfloat16),
            pltpu.VMEM((B, SQ, HD), jnp.bfloat16),
            pltpu.VMEM((B, SQ, HQ), jnp.float32),
            pltpu.VMEM((B, SQ, HQ), jnp.float32),
            pltpu.SemaphoreType.DMA((2, NC)),
            pltpu.SemaphoreType.DMA((2, NC)),
            pltpu.SemaphoreType.DMA((2, NC)),
            pltpu.SemaphoreType.DMA((NC,)),
        ],
        compiler_params=pltpu.CompilerParams(collective_id=0),
    )(x, Wq, K2, V2, Wo)


# device time: 16147 ns/iter; 1.2471x vs baseline; 1.0708x over previous
import jax
import jax.numpy as jnp
from jax import lax
from jax.experimental import pallas as pl
from jax.experimental.pallas import tpu as pltpu

N_DEV = 8
B = 2
SQ = 256
DMODEL = 512
DOUT = 512
HQ = 4
DH = 64
HD = HQ * DH
SKV_SH = 256
WIN = 128
REL1 = SQ + WIN - SKV_SH
SCALE = 0.125
RH = SQ // 2
CHUNKS = [(b, r) for b in range(B) for r in (0, RH)]
NC = len(CHUNKS)


def kernel(x, Wq, K_ext, V_ext, Wo):
    x = x.astype(jnp.bfloat16)
    Wq = Wq.astype(jnp.bfloat16)
    Wo = Wo.astype(jnp.bfloat16)
    K2 = K_ext.reshape(B, SKV_SH, HD).astype(jnp.bfloat16)
    V2 = V_ext.reshape(B, SKV_SH, HD).astype(jnp.bfloat16)

    def body(x_ref, wq_ref, k_ref, v_ref, wo_ref, out_ref,
             ctx_scr, ctx_rcv, l_snd, l_rcv, p1_send, p1_recv,
             bc_send, bc_recv):
        pos = lax.axis_index("i")
        peer = 1 - pos

        barrier = pltpu.get_barrier_semaphore()
        partner_sets = {0: (1, 3, 4), 1: (0, 2, 5), 2: (1, 6),
                        3: (0, 7), 4: (0,), 5: (1,), 6: (2,), 7: (3,)}
        for p, partners in partner_sets.items():
            @pl.when(pos == p)
            def _(partners=partners):
                for t in partners:
                    pl.semaphore_signal(
                        barrier, inc=1, device_id=(t,),
                        device_id_type=pl.DeviceIdType.MESH)
        n_partners = jnp.where(pos <= 1, 3, jnp.where(pos <= 3, 2, 1))
        pl.semaphore_wait(barrier, n_partners)

        def mask(r, kv_len, off):
            qi = lax.broadcasted_iota(jnp.int32, (RH, kv_len), 0) + r
            kj = lax.broadcasted_iota(jnp.int32, (RH, kv_len), 1) + off
            return jnp.abs(qi - kj) <= WIN

        def ctx_exchange_descr(c, src):
            b, r = CHUNKS[c]
            return pltpu.make_async_remote_copy(
                src_ref=src.at[b, pl.ds(r, RH)],
                dst_ref=ctx_rcv.at[b, pl.ds(r, RH)],
                send_sem=p1_send.at[0, c], recv_sem=p1_recv.at[0, c],
                device_id=(peer,), device_id_type=pl.DeviceIdType.MESH)

        def l_exchange_descr(c):
            b, r = CHUNKS[c]
            return pltpu.make_async_remote_copy(
                src_ref=l_snd.at[b, pl.ds(r, RH)],
                dst_ref=l_rcv.at[b, pl.ds(r, RH)],
                send_sem=p1_send.at[1, c], recv_sem=p1_recv.at[1, c],
                device_id=(peer,), device_id_type=pl.DeviceIdType.MESH)

        @pl.when(pos == 1)
        def _():
            keep = mask(RH, REL1, SKV_SH)
            for c, (b, r) in enumerate(CHUNKS):
                if r != RH:
                    continue
                q_bf = jnp.dot(x_ref[b, r:r + RH], wq_ref[...],
                               preferred_element_type=jnp.float32
                               ).astype(jnp.bfloat16)
                for h in range(HQ):
                    cs = slice(h * DH, (h + 1) * DH)
                    s = lax.dot_general(
                        q_bf[:, cs], k_ref[b, 0:REL1, cs],
                        (((1,), (1,)), ((), ())),
                        preferred_element_type=jnp.float32) * SCALE
                    w = jnp.where(keep, jnp.exp(s), 0.0)
                    l_snd[b, r:r + RH, h:h + 1] = jnp.sum(
                        w, axis=1, keepdims=True)
                    ctx_scr[b, r:r + RH, cs] = jnp.dot(
                        w.astype(jnp.bfloat16), v_ref[b, 0:REL1, cs],
                        preferred_element_type=jnp.float32
                        ).astype(jnp.bfloat16)
                xl = l_exchange_descr(c)
                xl.start()
                xc = ctx_exchange_descr(c, ctx_scr)
                xc.start()

        @pl.when(pos == 0)
        def _():
            keep0 = mask(0, SKV_SH, 0)
            keep1 = mask(RH, SKV_SH, 0)
            for b in range(B):
                q_bf = jnp.dot(x_ref[b], wq_ref[...],
                               preferred_element_type=jnp.float32
                               ).astype(jnp.bfloat16)
                for c, (cb, r) in enumerate(CHUNKS):
                    if cb != b:
                        continue
                    keep = keep0 if r == 0 else keep1
                    rs = pl.ds(r, RH)
                    for h in range(HQ):
                        cs = slice(h * DH, (h + 1) * DH)
                        s = lax.dot_general(
                            q_bf[r:r + RH, cs], k_ref[b, :, cs],
                            (((1,), (1,)), ((), ())),
                            preferred_element_type=jnp.float32) * SCALE
                        w = jnp.where(keep, jnp.exp(s), 0.0)
                        if r == 0:
                            rcp = 1.0 / jnp.sum(w, axis=1, keepdims=True)
                            ctx_rcv[b, rs, cs] = jnp.dot(
                                (w * rcp).astype(jnp.bfloat16),
                                v_ref[b, :, cs],
                                preferred_element_type=jnp.float32
                                ).astype(jnp.bfloat16)
                        else:
                            l_snd[b, rs, h:h + 1] = jnp.sum(
                                w, axis=1, keepdims=True)
                            ctx_scr[b, rs, cs] = jnp.dot(
                                w.astype(jnp.bfloat16), v_ref[b, :, cs],
                                preferred_element_type=jnp.float32
                                ).astype(jnp.bfloat16)
                    if r == 0:
                        xc = ctx_exchange_descr(c, ctx_rcv)
                        xc.start()
                    else:
                        xl = l_exchange_descr(c)
                        xl.start()
                        xc = ctx_exchange_descr(c, ctx_scr)
                        xc.start()

        has_recv = pos >= 2
        is_fwd = jnp.logical_or(pos == 2, pos == 3)
        recv_from = jnp.where(
            jnp.logical_or(pos == 3, pos == 4), 0,
            jnp.where(jnp.logical_or(pos == 2, pos == 5), 1,
                      jnp.where(pos == 6, 2, 3)))
        child0 = jnp.where(pos == 0, 3,
                           jnp.where(pos == 1, 2,
                                     jnp.where(pos == 2, 6, 7)))
        child1 = jnp.where(pos == 0, 4, 5)

        def bc_descr(c, sender_slot, target, src):
            b, r = CHUNKS[c]
            return pltpu.make_async_remote_copy(
                src_ref=src.at[b, pl.ds(r, RH)],
                dst_ref=ctx_scr.at[b, pl.ds(r, RH)],
                send_sem=bc_send.at[sender_slot, c],
                recv_sem=bc_recv.at[c],
                device_id=(target,), device_id_type=pl.DeviceIdType.MESH)

        for c, (b, r) in enumerate(CHUNKS):
            rs = pl.ds(r, RH)

            if r == RH:
                @pl.when(pos <= 1)
                def _():
                    l_exchange_descr(c).wait_recv()
                    ctx_exchange_descr(c, ctx_scr).wait_recv()
                    for h in range(HQ):
                        cs = slice(h * DH, (h + 1) * DH)
                        lt = l_snd[b, rs, h:h + 1] + l_rcv[b, rs, h:h + 1]
                        ctx_rcv[b, rs, cs] = (
                            (ctx_scr[b, rs, cs].astype(jnp.float32)
                             + ctx_rcv[b, rs, cs].astype(jnp.float32))
                            / lt).astype(jnp.bfloat16)
            else:
                @pl.when(pos == 1)
                def _():
                    ctx_exchange_descr(c, ctx_scr).wait_recv()

            @pl.when(has_recv)
            def _():
                bc_descr(c, 0, recv_from, ctx_scr).wait_recv()

            @pl.when(pos <= 1)
            def _():
                bc_descr(c, 0, child0, ctx_rcv).start()
                bc_descr(c, 1, child1, ctx_rcv).start()

            @pl.when(is_fwd)
            def _():
                bc_descr(c, 0, child0, ctx_scr).start()

            @pl.when(has_recv)
            def _():
                out_ref[b, r:r + RH] = jnp.dot(
                    ctx_scr[b, r:r + RH], wo_ref[...],
                    preferred_element_type=jnp.float32)

        @pl.when(pos <= 1)
        def _():
            for b in range(B):
                out_ref[b] = jnp.dot(ctx_rcv[b], wo_ref[...],
                                     preferred_element_type=jnp.float32)

        for c, (b, r) in enumerate(CHUNKS):
            @pl.when(pos == 0)
            def _():
                if r == 0:
                    ctx_exchange_descr(c, ctx_rcv).wait_send()
                else:
                    l_exchange_descr(c).wait_send()
                    ctx_exchange_descr(c, ctx_scr).wait_send()

            if r == RH:
                @pl.when(pos == 1)
                def _():
                    l_exchange_descr(c).wait_send()
                    ctx_exchange_descr(c, ctx_scr).wait_send()

            @pl.when(pos <= 1)
            def _():
                bc_descr(c, 0, child0, ctx_rcv).wait_send()
                bc_descr(c, 1, child1, ctx_rcv).wait_send()

            @pl.when(is_fwd)
            def _():
                bc_descr(c, 0, child0, ctx_scr).wait_send()

    return pl.pallas_call(
        body,
        out_shape=jax.ShapeDtypeStruct((B, SQ, DOUT), jnp.float32),
        in_specs=[pl.BlockSpec(memory_space=pltpu.VMEM)] * 5,
        out_specs=pl.BlockSpec(memory_space=pltpu.VMEM),
        scratch_shapes=[
            pltpu.VMEM((B, SQ, HD), jnp.bfloat16),
            pltpu.VMEM((B, SQ, HD), jnp.bfloat16),
            pltpu.VMEM((B, SQ, HQ), jnp.float32),
            pltpu.VMEM((B, SQ, HQ), jnp.float32),
            pltpu.SemaphoreType.DMA((2, NC)),
            pltpu.SemaphoreType.DMA((2, NC)),
            pltpu.SemaphoreType.DMA((2, NC)),
            pltpu.SemaphoreType.DMA((NC,)),
        ],
        compiler_params=pltpu.CompilerParams(collective_id=0),
    )(x, Wq, K2, V2, Wo)
